# baseline (speedup 1.0000x reference)
import functools

import jax
import jax.numpy as jnp
from jax import lax
from jax.experimental import pallas as pl
from jax.experimental.pallas import tpu as pltpu
from jax.experimental.pallas import tpu_sc as plsc

VOCAB = 1000000
EMBED = 64
B = 16384
NEG = 20

NC, NS, L = 2, 16, 16
NW = NC * NS
BPW = B // NW
CB = 32
NCHUNK = BPW // CB
NIDX_ROWS = CB * NEG // 128


def _sc_body(context_hbm, cemb_hbm, pos_hbm, neg_hbm,
             pos_out, neg_out,
             pidx0, nidx0, c0, p0, n0,
             pidx1, nidx1, c1, p1, n1,
             pos_sv, neg_sv, sem0, sem1):
    wid = lax.axis_index("s") * NC + lax.axis_index("c")
    lane = lax.iota(jnp.int32, L)
    last_lane = lane == (L - 1)
    set0 = (pidx0, nidx0, c0, p0, n0, sem0)
    set1 = (pidx1, nidx1, c1, p1, n1, sem1)

    def issue(ch, bufs):
        pidx, nidx, c_rows, p_rows, n_rows, sem = bufs
        base = wid * BPW + ch * CB
        pltpu.sync_copy(pos_hbm.at[pl.ds(base, CB)], pidx)
        pltpu.sync_copy(neg_hbm.at[pl.ds(base * NEG, CB * NEG)], nidx)
        pltpu.async_copy(cemb_hbm.at[pl.ds(base, CB), :], c_rows, sem)
        pltpu.async_copy(context_hbm.at[pidx], p_rows, sem)
        for j in range(NIDX_ROWS):
            pltpu.async_copy(context_hbm.at[nidx.at[pl.ds(j * 128, 128)]],
                             n_rows.at[pl.ds(j * 128, 128), :], sem)

    def wait_bufs(bufs):
        pidx, nidx, c_rows, p_rows, n_rows, sem = bufs
        pltpu.make_async_copy(cemb_hbm.at[pl.ds(0, CB), :], c_rows, sem).wait()
        pltpu.make_async_copy(cemb_hbm.at[pl.ds(0, CB), :], p_rows, sem).wait()
        pltpu.make_async_copy(cemb_hbm.at[pl.ds(0, CB * NEG), :],
                              n_rows, sem).wait()

    perms = tuple(lane ^ st for st in (8, 4, 2, 1))

    def _allsum(s):
        for perm in perms:
            s = s + jnp.take_along_axis(s, perm, axis=0)
        return s

    def compute(ch, bufs):
        _, _, c_rows, p_rows, n_rows, _ = bufs
        base = wid * BPW + ch * CB
        zero = jnp.zeros((L,), jnp.float32)

        def group_body(g, carry2):
            def elem_body(ii, acc):
                accp, accn = acc
                i = g * L + ii
                cvs = tuple(c_rows[i, pl.ds(db * L, L)]
                            for db in range(EMBED // L))
                pvs = tuple(p_rows[i, pl.ds(db * L, L)]
                            for db in range(EMBED // L))
                s = cvs[0] * pvs[0]
                for db in range(1, EMBED // L):
                    s = s + cvs[db] * pvs[db]
                sel = lane == ii
                accp = jnp.where(sel, _allsum(s), accp)
                new_accn = []
                for k in range(NEG):
                    r = i * NEG + k
                    s2 = n_rows[r, pl.ds(0, L)] * cvs[0]
                    for db in range(1, EMBED // L):
                        s2 = s2 + n_rows[r, pl.ds(db * L, L)] * cvs[db]
                    new_accn.append(jnp.where(sel, _allsum(s2), accn[k]))
                return accp, tuple(new_accn)

            accp, accn = lax.fori_loop(0, L, elem_body,
                                       (zero, (zero,) * NEG))
            pos_sv[pl.ds(g * L, L)] = accp
            grow = (g * L + lane) * NEG
            for k in range(NEG):
                plsc.store_scatter(neg_sv, [grow + k], accn[k])
            return carry2

        lax.fori_loop(0, CB // L, group_body, 0)
        pltpu.sync_copy(pos_sv, pos_out.at[pl.ds(base, CB)])
        pltpu.sync_copy(neg_sv, neg_out.at[pl.ds(base * NEG, CB * NEG)])

    issue(0, set0)

    def pair_body(h, carry):
        ch0 = 2 * h
        issue(ch0 + 1, set1)
        wait_bufs(set0)
        compute(ch0, set0)
        issue(lax.min(ch0 + 2, NCHUNK - 1), set0)
        wait_bufs(set1)
        compute(ch0 + 1, set1)
        return carry

    lax.fori_loop(0, NCHUNK // 2, pair_body, 0)
    wait_bufs(set0)


@functools.lru_cache(maxsize=None)
def _build_sc_scores():
  return functools.partial(
    pl.kernel,
    out_type=(jax.ShapeDtypeStruct((B,), jnp.float32),
              jax.ShapeDtypeStruct((B * NEG,), jnp.float32)),
    mesh=plsc.VectorSubcoreMesh(core_axis_name="c", subcore_axis_name="s",
                                num_cores=NC, num_subcores=NS),
    compiler_params=pltpu.CompilerParams(needs_layout_passes=False,
                                         use_tc_tiling_on_sc=False),
    scratch_types=(
        [pltpu.VMEM((CB,), jnp.int32),
         pltpu.VMEM((CB * NEG,), jnp.int32),
         pltpu.VMEM((CB, EMBED), jnp.float32),
         pltpu.VMEM((CB, EMBED), jnp.float32),
         pltpu.VMEM((CB * NEG, EMBED), jnp.float32)] * 2
        + [pltpu.VMEM((CB,), jnp.float32),
           pltpu.VMEM((CB * NEG,), jnp.float32),
           pltpu.SemaphoreType.DMA,
           pltpu.SemaphoreType.DMA]
    ),
  )(_sc_body)


def _loss_body(pos_ref, neg_ref, out_ref):
    eps = 1e-07
    ps = pos_ref[...]
    ns = neg_ref[...]
    pos_loss = -jnp.log(jax.nn.sigmoid(ps) + eps)
    neg_loss = -jnp.log(jax.nn.sigmoid(-ns) + eps)
    out_ref[0, 0] = (jnp.sum(pos_loss) / float(B)
                     + jnp.sum(neg_loss) / float(B * NEG))


_tc_loss = pl.pallas_call(
    _loss_body,
    out_shape=jax.ShapeDtypeStruct((1, 1), jnp.float32),
    out_specs=pl.BlockSpec(memory_space=pltpu.SMEM),
)


def kernel(c, pos, neg, center_w, context_w):
    c = c.astype(jnp.int32)
    pos = pos.astype(jnp.int32)
    negf = neg.astype(jnp.int32).reshape(B * NEG)
    ctx_probe = jnp.take(context_w, jnp.zeros((8,), jnp.int32), axis=0)
    c2 = c + (ctx_probe[0, 0] * 0.0).astype(jnp.int32)
    c_emb = jnp.take(center_w, c2, axis=0)
    pos_s, neg_s = _build_sc_scores()(context_w, c_emb, pos, negf)
    loss = _tc_loss(pos_s.reshape(B // 128, 128),
                    neg_s.reshape(B * NEG // 128, 128))
    return loss[0, 0]


# speedup vs baseline: 1.0173x; 1.0173x over previous
import functools

import jax
import jax.numpy as jnp
from jax import lax
from jax.experimental import pallas as pl
from jax.experimental.pallas import tpu as pltpu
from jax.experimental.pallas import tpu_sc as plsc

VOCAB = 1000000
EMBED = 64
B = 16384
NEG = 20

NC, NS, L = 2, 16, 16
NW = NC * NS
BPW = B // NW
CB = 32
NCHUNK = BPW // CB
NIDX_ROWS = CB * NEG // 128


def _sc_body(context_hbm, cemb_hbm, pos_hbm, neg_hbm,
             pos_out, neg_out,
             pidx0, nidx0, c0, p0, n0,
             pidx1, nidx1, c1, p1, n1,
             pos_sv, neg_sv, sem0, sem1):
    wid = lax.axis_index("s") * NC + lax.axis_index("c")
    lane = lax.iota(jnp.int32, L)
    last_lane = lane == (L - 1)
    set0 = (pidx0, nidx0, c0, p0, n0, sem0)
    set1 = (pidx1, nidx1, c1, p1, n1, sem1)

    def issue(ch, bufs):
        pidx, nidx, c_rows, p_rows, n_rows, sem = bufs
        base = wid * BPW + ch * CB
        pltpu.sync_copy(pos_hbm.at[pl.ds(base, CB)], pidx)
        pltpu.sync_copy(neg_hbm.at[pl.ds(base * NEG, CB * NEG)], nidx)
        pltpu.async_copy(cemb_hbm.at[pl.ds(base, CB), :], c_rows, sem)
        pltpu.async_copy(context_hbm.at[pidx], p_rows, sem)
        for j in range(NIDX_ROWS):
            pltpu.async_copy(context_hbm.at[nidx.at[pl.ds(j * 128, 128)]],
                             n_rows.at[pl.ds(j * 128, 128), :], sem)

    def wait_bufs(bufs):
        pidx, nidx, c_rows, p_rows, n_rows, sem = bufs
        pltpu.make_async_copy(cemb_hbm.at[pl.ds(0, CB), :], c_rows, sem).wait()
        pltpu.make_async_copy(cemb_hbm.at[pl.ds(0, CB), :], p_rows, sem).wait()
        pltpu.make_async_copy(cemb_hbm.at[pl.ds(0, CB * NEG), :],
                              n_rows, sem).wait()

    perms = tuple(lane ^ st for st in (8, 4, 2, 1))

    def _allsum(s):
        for perm in perms:
            s = s + jnp.take_along_axis(s, perm, axis=0)
        return s

    def compute(ch, bufs):
        _, _, c_rows, p_rows, n_rows, _ = bufs
        base = wid * BPW + ch * CB
        zero = jnp.zeros((L,), jnp.float32)

        def group_body(g, carry2):
            def elem_body(ii, acc):
                accp, accn = acc
                i = g * L + ii
                cvs = tuple(c_rows[i, pl.ds(db * L, L)]
                            for db in range(EMBED // L))
                pvs = tuple(p_rows[i, pl.ds(db * L, L)]
                            for db in range(EMBED // L))
                s = cvs[0] * pvs[0]
                for db in range(1, EMBED // L):
                    s = s + cvs[db] * pvs[db]
                sel = lane == ii
                accp = jnp.where(sel, _allsum(s), accp)
                new_accn = []
                for k in range(NEG):
                    r = i * NEG + k
                    s2 = n_rows[r, pl.ds(0, L)] * cvs[0]
                    for db in range(1, EMBED // L):
                        s2 = s2 + n_rows[r, pl.ds(db * L, L)] * cvs[db]
                    new_accn.append(jnp.where(sel, _allsum(s2), accn[k]))
                return accp, tuple(new_accn)

            accp, accn = lax.fori_loop(0, L, elem_body,
                                       (zero, (zero,) * NEG))
            pos_sv[pl.ds(g * L, L)] = accp
            grow = (g * L + lane) * NEG
            for k in range(NEG):
                plsc.store_scatter(neg_sv, [grow + k], accn[k])
            return carry2

        lax.fori_loop(0, CB // L, group_body, 0)
        pltpu.sync_copy(pos_sv, pos_out.at[pl.ds(base, CB)])
        pltpu.sync_copy(neg_sv, neg_out.at[pl.ds(base * NEG, CB * NEG)])

    issue(0, set0)

    def pair_body(h, carry):
        ch0 = 2 * h
        issue(ch0 + 1, set1)
        wait_bufs(set0)
        compute(ch0, set0)
        issue(lax.min(ch0 + 2, NCHUNK - 1), set0)
        wait_bufs(set1)
        compute(ch0 + 1, set1)
        return carry

    lax.fori_loop(0, NCHUNK // 2, pair_body, 0)
    wait_bufs(set0)


@functools.lru_cache(maxsize=None)
def _build_sc_scores():
  return functools.partial(
    pl.kernel,
    out_type=(jax.ShapeDtypeStruct((B,), jnp.float32),
              jax.ShapeDtypeStruct((B * NEG,), jnp.float32)),
    mesh=plsc.VectorSubcoreMesh(core_axis_name="c", subcore_axis_name="s",
                                num_cores=NC, num_subcores=NS),
    compiler_params=pltpu.CompilerParams(needs_layout_passes=False,
                                         use_tc_tiling_on_sc=False),
    scratch_types=(
        [pltpu.VMEM((CB,), jnp.int32),
         pltpu.VMEM((CB * NEG,), jnp.int32),
         pltpu.VMEM((CB, EMBED), jnp.float32),
         pltpu.VMEM((CB, EMBED), jnp.float32),
         pltpu.VMEM((CB * NEG, EMBED), jnp.float32)] * 2
        + [pltpu.VMEM((CB,), jnp.float32),
           pltpu.VMEM((CB * NEG,), jnp.float32),
           pltpu.SemaphoreType.DMA,
           pltpu.SemaphoreType.DMA]
    ),
  )(_sc_body)


def _loss_body(pos_ref, neg_ref, out_ref):
    eps = 1e-07
    ps = pos_ref[...]
    ns = neg_ref[...]
    pos_loss = -jnp.log(jax.nn.sigmoid(ps) + eps)
    neg_loss = -jnp.log(jax.nn.sigmoid(-ns) + eps)
    out_ref[0, 0] = (jnp.sum(pos_loss) / float(B)
                     + jnp.sum(neg_loss) / float(B * NEG))


_tc_loss = pl.pallas_call(
    _loss_body,
    out_shape=jax.ShapeDtypeStruct((1, 1), jnp.float32),
    out_specs=pl.BlockSpec(memory_space=pltpu.SMEM),
)


def kernel(c, pos, neg, center_w, context_w):
    c = c.astype(jnp.int32)
    pos = pos.astype(jnp.int32)
    negf = neg.astype(jnp.int32).reshape(B * NEG)
    c_emb = jnp.take(center_w, c, axis=0)
    pos_s, neg_s = _build_sc_scores()(context_w, c_emb, pos, negf)
    loss = _tc_loss(pos_s.reshape(B // 128, 128),
                    neg_s.reshape(B * NEG // 128, 128))
    return loss[0, 0]
